# Initial kernel scaffold; baseline (speedup 1.0000x reference)
import jax
import jax.numpy as jnp
from jax.experimental import pallas as pl


def kernel(z, edge):
    raise NotImplementedError("write your pallas kernel here")


# speedup vs baseline: 2.7915x; 2.7915x over previous
import functools

import jax
import jax.numpy as jnp
from jax import lax
from jax.experimental import pallas as pl
from jax.experimental.pallas import tpu as pltpu
from jax.experimental.pallas import tpu_sc as plsc

_CHUNK = 128
_GROUP = 16
_D = 128


def _lane_gather(a, perm):
    dnums = lax.GatherDimensionNumbers(
        offset_dims=(), collapsed_slice_dims=(0,), start_index_map=(0,))
    return lax.gather(a, perm[:, None], dnums, (1,),
                      mode=lax.GatherScatterMode.PROMISE_IN_BOUNDS)


@functools.cache
def _build(n_edges, n_rows, interpret=False):
    info = plsc.get_sparse_core_info()
    nc, ns = info.num_cores, info.num_subcores
    nw = nc * ns
    n_chunks = n_edges // _CHUNK
    assert n_edges % _CHUNK == 0

    mesh = plsc.VectorSubcoreMesh(core_axis_name="c", subcore_axis_name="s")

    @functools.partial(
        pl.kernel,
        mesh=mesh,
        out_type=jax.ShapeDtypeStruct((n_edges,), jnp.float32),
        scratch_types=[
            pltpu.VMEM((_CHUNK,), jnp.int32),
            pltpu.VMEM((_CHUNK,), jnp.int32),
            pltpu.VMEM((_CHUNK, _D), jnp.float32),
            pltpu.VMEM((_CHUNK, _D), jnp.float32),
            pltpu.VMEM((_CHUNK,), jnp.float32),
            pltpu.SemaphoreType.DMA,
        ],
    )
    def k(z_hbm, e0_hbm, e1_hbm, out_hbm, idx0_v, idx1_v, rows0_v, rows1_v,
          out_v, sem):
        wid = lax.axis_index("s") * nc + lax.axis_index("c")
        n_loc = (n_chunks - wid + nw - 1) // nw

        lane = lax.iota(jnp.int32, 16)
        perms = {k_: lane ^ k_ for k_ in (1, 2, 4, 8)}
        masks = {k_: (lane & k_) != 0 for k_ in (1, 2, 4, 8)}

        def chunk_body(i, carry):
            base = (wid + i * nw) * _CHUNK
            pltpu.sync_copy(e0_hbm.at[pl.ds(base, _CHUNK)], idx0_v)
            pltpu.sync_copy(e1_hbm.at[pl.ds(base, _CHUNK)], idx1_v)
            c0 = pltpu.async_copy(z_hbm.at[idx0_v], rows0_v, sem)
            c1 = pltpu.async_copy(z_hbm.at[idx1_v], rows1_v, sem)
            c0.wait()
            c1.wait()

            def group_body(g, carry2):
                e_base = g * _GROUP
                vecs = []
                for j in range(_GROUP):
                    e = e_base + j
                    acc = (rows0_v[e, pl.ds(0, 16)] *
                           rows1_v[e, pl.ds(0, 16)])
                    for kk in range(1, _D // 16):
                        acc = acc + (rows0_v[e, pl.ds(kk * 16, 16)] *
                                     rows1_v[e, pl.ds(kk * 16, 16)])
                    vecs.append(acc)
                for k_ in (1, 2, 4, 8):
                    nxt = []
                    for p in range(0, len(vecs), 2):
                        a, b = vecs[p], vecs[p + 1]
                        ra = a + _lane_gather(a, perms[k_])
                        rb = b + _lane_gather(b, perms[k_])
                        nxt.append(jnp.where(masks[k_], rb, ra))
                    vecs = nxt
                x = vecs[0]
                y = 1.0 / (1.0 + jnp.exp(-x))
                out_v[pl.ds(e_base, 16)] = y
                return carry2

            lax.fori_loop(0, _CHUNK // _GROUP, group_body, 0)
            pltpu.sync_copy(out_v, out_hbm.at[pl.ds(base, _CHUNK)])
            return carry

        lax.fori_loop(0, n_loc, chunk_body, 0)

    return k


def kernel(z, edge):
    z = z.astype(jnp.float32)
    e0 = edge[0].astype(jnp.int32)
    e1 = edge[1].astype(jnp.int32)
    k = _build(e0.shape[0], z.shape[0])
    return k(z, e0, e1)


# speedup vs baseline: 4.2747x; 1.5313x over previous
import functools

import jax
import jax.numpy as jnp
from jax import lax
from jax.experimental import pallas as pl
from jax.experimental.pallas import tpu as pltpu
from jax.experimental.pallas import tpu_sc as plsc

_CHUNK = 80
_GROUP = 16
_D = 128


def _lane_gather(a, perm):
    dnums = lax.GatherDimensionNumbers(
        offset_dims=(), collapsed_slice_dims=(0,), start_index_map=(0,))
    return lax.gather(a, perm[:, None], dnums, (1,),
                      mode=lax.GatherScatterMode.PROMISE_IN_BOUNDS)


@functools.cache
def _build(n_edges, n_rows):
    info = plsc.get_sparse_core_info()
    nc, ns = info.num_cores, info.num_subcores
    nw = nc * ns
    b_per_w = n_edges // nw
    n_chunks = b_per_w // _CHUNK
    assert n_edges % nw == 0 and b_per_w % _CHUNK == 0 and n_chunks % 2 == 1

    mesh = plsc.VectorSubcoreMesh(core_axis_name="c", subcore_axis_name="s")

    @functools.partial(
        pl.kernel,
        mesh=mesh,
        out_type=jax.ShapeDtypeStruct((n_edges,), jnp.float32),
        scratch_types=[
            pltpu.VMEM((b_per_w,), jnp.int32),
            pltpu.VMEM((b_per_w,), jnp.int32),
            pltpu.VMEM((_CHUNK, _D), jnp.float32),
            pltpu.VMEM((_CHUNK, _D), jnp.float32),
            pltpu.VMEM((_CHUNK, _D), jnp.float32),
            pltpu.VMEM((_CHUNK, _D), jnp.float32),
            pltpu.VMEM((b_per_w,), jnp.float32),
            pltpu.SemaphoreType.DMA,
            pltpu.SemaphoreType.DMA,
            pltpu.SemaphoreType.DMA,
        ],
    )
    def k(z_hbm, e0_hbm, e1_hbm, out_hbm, idx0_v, idx1_v, rows0a_v, rows1a_v,
          rows0b_v, rows1b_v, out_v, isem, gsem_a, gsem_b):
        wid = lax.axis_index("s") * nc + lax.axis_index("c")
        base = wid * b_per_w

        lane = lax.iota(jnp.int32, 16)
        perms = {k_: lane ^ k_ for k_ in (1, 2, 4, 8)}
        masks = {k_: (lane & k_) != 0 for k_ in (1, 2, 4, 8)}

        ci0 = pltpu.async_copy(e0_hbm.at[pl.ds(base, b_per_w)], idx0_v, isem)
        ci1 = pltpu.async_copy(e1_hbm.at[pl.ds(base, b_per_w)], idx1_v, isem)
        ci0.wait()
        ci1.wait()

        def launch(c, r0, r1, sem):
            g0 = pltpu.async_copy(
                z_hbm.at[idx0_v.at[pl.ds(c * _CHUNK, _CHUNK)]], r0, sem)
            g1 = pltpu.async_copy(
                z_hbm.at[idx1_v.at[pl.ds(c * _CHUNK, _CHUNK)]], r1, sem)
            return g0, g1

        def drain(c, r0, r1, sem):
            pltpu.make_async_copy(
                z_hbm.at[idx0_v.at[pl.ds(c * _CHUNK, _CHUNK)]], r0, sem).wait()
            pltpu.make_async_copy(
                z_hbm.at[idx1_v.at[pl.ds(c * _CHUNK, _CHUNK)]], r1, sem).wait()

        def compute(c, r0, r1):
            def group_body(g, carry):
                vecs = []
                for j in range(_GROUP):
                    e = g * _GROUP + j
                    acc = r0[e, pl.ds(0, 16)] * r1[e, pl.ds(0, 16)]
                    for kk in range(1, _D // 16):
                        acc = acc + (r0[e, pl.ds(kk * 16, 16)] *
                                     r1[e, pl.ds(kk * 16, 16)])
                    vecs.append(acc)
                for k_ in (1, 2, 4, 8):
                    nxt = []
                    for p in range(0, len(vecs), 2):
                        a, b = vecs[p], vecs[p + 1]
                        ra = a + _lane_gather(a, perms[k_])
                        rb = b + _lane_gather(b, perms[k_])
                        nxt.append(jnp.where(masks[k_], rb, ra))
                    vecs = nxt
                x = vecs[0]
                y = 1.0 / (1.0 + jnp.exp(-x))
                out_v[pl.ds(c * _CHUNK + g * _GROUP, _GROUP)] = y
                return carry

            lax.fori_loop(0, _CHUNK // _GROUP, group_body, 0)

        launch(0, rows0a_v, rows1a_v, gsem_a)

        def pair_body(it, carry):
            c = it * 2
            launch(c + 1, rows0b_v, rows1b_v, gsem_b)
            drain(c, rows0a_v, rows1a_v, gsem_a)
            compute(c, rows0a_v, rows1a_v)
            launch(c + 2, rows0a_v, rows1a_v, gsem_a)
            drain(c + 1, rows0b_v, rows1b_v, gsem_b)
            compute(c + 1, rows0b_v, rows1b_v)
            return carry

        lax.fori_loop(0, (n_chunks - 1) // 2, pair_body, 0)

        drain(n_chunks - 1, rows0a_v, rows1a_v, gsem_a)
        compute(n_chunks - 1, rows0a_v, rows1a_v)

        pltpu.sync_copy(out_v, out_hbm.at[pl.ds(base, b_per_w)])

    return k


def kernel(z, edge):
    z = z.astype(jnp.float32)
    e0 = edge[0].astype(jnp.int32)
    e1 = edge[1].astype(jnp.int32)
    k = _build(e0.shape[0], z.shape[0])
    return k(z, e0, e1)


# speedup vs baseline: 9.4751x; 2.2165x over previous
import functools

import jax
import jax.numpy as jnp
from jax import lax
from jax.experimental import pallas as pl
from jax.experimental.pallas import tpu as pltpu
from jax.experimental.pallas import tpu_sc as plsc

_CHUNK = 80
_GROUP = 16
_D = 128
_DW = _D // 2


def _lane_gather(a, perm):
    dnums = lax.GatherDimensionNumbers(
        offset_dims=(), collapsed_slice_dims=(0,), start_index_map=(0,))
    return lax.gather(a, perm[:, None], dnums, (1,),
                      mode=lax.GatherScatterMode.PROMISE_IN_BOUNDS)


@functools.cache
def _build(n_edges, n_rows):
    info = plsc.get_sparse_core_info()
    nc, ns = info.num_cores, info.num_subcores
    nw = nc * ns
    b_per_w = n_edges // nw
    n_chunks = b_per_w // _CHUNK
    assert n_edges % nw == 0 and b_per_w % _CHUNK == 0 and n_chunks % 2 == 1

    mesh = plsc.VectorSubcoreMesh(core_axis_name="c", subcore_axis_name="s")

    @functools.partial(
        pl.kernel,
        mesh=mesh,
        compiler_params=pltpu.CompilerParams(use_tc_tiling_on_sc=False),
        out_type=jax.ShapeDtypeStruct((n_edges,), jnp.float32),
        scratch_types=[
            pltpu.VMEM((b_per_w,), jnp.int32),
            pltpu.VMEM((b_per_w,), jnp.int32),
            pltpu.VMEM((_CHUNK, _DW), jnp.int32),
            pltpu.VMEM((_CHUNK, _DW), jnp.int32),
            pltpu.VMEM((_CHUNK, _DW), jnp.int32),
            pltpu.VMEM((_CHUNK, _DW), jnp.int32),
            pltpu.VMEM((b_per_w,), jnp.float32),
            pltpu.SemaphoreType.DMA,
            pltpu.SemaphoreType.DMA,
            pltpu.SemaphoreType.DMA,
        ],
    )
    def k(z_hbm, e0_hbm, e1_hbm, out_hbm, idx0_v, idx1_v, rows0a_v, rows1a_v,
          rows0b_v, rows1b_v, out_v, isem, gsem_a, gsem_b):
        wid = lax.axis_index("s") * nc + lax.axis_index("c")
        base = wid * b_per_w

        lane = lax.iota(jnp.int32, 16)
        perms = {k_: lane ^ k_ for k_ in (1, 2, 4, 8)}
        masks = {k_: (lane & k_) != 0 for k_ in (1, 2, 4, 8)}

        ci0 = pltpu.async_copy(e0_hbm.at[pl.ds(base, b_per_w)], idx0_v, isem)
        ci1 = pltpu.async_copy(e1_hbm.at[pl.ds(base, b_per_w)], idx1_v, isem)
        ci0.wait()
        ci1.wait()

        def launch(c, r0, r1, sem):
            pltpu.async_copy(
                z_hbm.at[idx0_v.at[pl.ds(c * _CHUNK, _CHUNK)]], r0, sem)
            pltpu.async_copy(
                z_hbm.at[idx1_v.at[pl.ds(c * _CHUNK, _CHUNK)]], r1, sem)

        def drain(c, r0, r1, sem):
            pltpu.make_async_copy(
                z_hbm.at[idx0_v.at[pl.ds(c * _CHUNK, _CHUNK)]], r0, sem).wait()
            pltpu.make_async_copy(
                z_hbm.at[idx1_v.at[pl.ds(c * _CHUNK, _CHUNK)]], r1, sem).wait()

        himask = jnp.int32(-65536)

        def dot16(r0, r1, e):
            acc = None
            for kk in range(_DW // 16):
                ws = r0[e, pl.ds(kk * 16, 16)]
                wd = r1[e, pl.ds(kk * 16, 16)]
                hs = lax.bitcast_convert_type(ws & himask, jnp.float32)
                hd = lax.bitcast_convert_type(wd & himask, jnp.float32)
                ls = lax.bitcast_convert_type(ws << 16, jnp.float32)
                ld = lax.bitcast_convert_type(wd << 16, jnp.float32)
                s = hs * hd + ls * ld
                acc = s if acc is None else acc + s
            return acc

        def compute(c, r0, r1):
            def group_body(g, carry):
                vecs = [dot16(r0, r1, g * _GROUP + j) for j in range(_GROUP)]
                for k_ in (1, 2, 4, 8):
                    nxt = []
                    for p in range(0, len(vecs), 2):
                        a, b = vecs[p], vecs[p + 1]
                        ra = a + _lane_gather(a, perms[k_])
                        rb = b + _lane_gather(b, perms[k_])
                        nxt.append(jnp.where(masks[k_], rb, ra))
                    vecs = nxt
                x = vecs[0]
                y = 1.0 / (1.0 + jnp.exp(-x))
                out_v[pl.ds(c * _CHUNK + g * _GROUP, _GROUP)] = y
                return carry

            lax.fori_loop(0, _CHUNK // _GROUP, group_body, 0)

        launch(0, rows0a_v, rows1a_v, gsem_a)

        def pair_body(it, carry):
            c = it * 2
            launch(c + 1, rows0b_v, rows1b_v, gsem_b)
            drain(c, rows0a_v, rows1a_v, gsem_a)
            compute(c, rows0a_v, rows1a_v)
            launch(c + 2, rows0a_v, rows1a_v, gsem_a)
            drain(c + 1, rows0b_v, rows1b_v, gsem_b)
            compute(c + 1, rows0b_v, rows1b_v)
            return carry

        lax.fori_loop(0, (n_chunks - 1) // 2, pair_body, 0)

        drain(n_chunks - 1, rows0a_v, rows1a_v, gsem_a)
        compute(n_chunks - 1, rows0a_v, rows1a_v)

        pltpu.sync_copy(out_v, out_hbm.at[pl.ds(base, b_per_w)])

    return k


def kernel(z, edge):
    z16 = z.astype(jnp.bfloat16)
    zw = lax.bitcast_convert_type(z16.reshape(z.shape[0], _DW, 2), jnp.int32)
    e0 = edge[0].astype(jnp.int32)
    e1 = edge[1].astype(jnp.int32)
    k = _build(e0.shape[0], z.shape[0])
    return k(zw, e0, e1)


# speedup vs baseline: 9.7964x; 1.0339x over previous
import functools

import jax
import jax.numpy as jnp
from jax import lax
from jax.experimental import pallas as pl
from jax.experimental.pallas import tpu as pltpu
from jax.experimental.pallas import tpu_sc as plsc

_CHUNK = 80
_GROUP = 16
_D = 128
_DW = _D // 2


def _lane_gather(a, perm):
    dnums = lax.GatherDimensionNumbers(
        offset_dims=(), collapsed_slice_dims=(0,), start_index_map=(0,))
    return lax.gather(a, perm[:, None], dnums, (1,),
                      mode=lax.GatherScatterMode.PROMISE_IN_BOUNDS)


@functools.cache
def _build(n_edges, n_rows):
    info = plsc.get_sparse_core_info()
    nc, ns = info.num_cores, info.num_subcores
    nw = nc * ns
    b_per_w = n_edges // nw
    n_chunks = b_per_w // _CHUNK
    assert n_edges % nw == 0 and b_per_w % _CHUNK == 0 and n_chunks % 2 == 1

    mesh = plsc.VectorSubcoreMesh(core_axis_name="c", subcore_axis_name="s")

    @functools.partial(
        pl.kernel,
        mesh=mesh,
        compiler_params=pltpu.CompilerParams(use_tc_tiling_on_sc=False),
        out_type=jax.ShapeDtypeStruct((n_edges,), jnp.float32),
        scratch_types=[
            pltpu.VMEM((b_per_w,), jnp.int32),
            pltpu.VMEM((b_per_w,), jnp.int32),
            pltpu.VMEM((_CHUNK, _DW), jnp.int32),
            pltpu.VMEM((_CHUNK, _DW), jnp.int32),
            pltpu.VMEM((_CHUNK, _DW), jnp.int32),
            pltpu.VMEM((_CHUNK, _DW), jnp.int32),
            pltpu.VMEM((b_per_w,), jnp.float32),
            pltpu.SemaphoreType.DMA,
            pltpu.SemaphoreType.DMA,
            pltpu.SemaphoreType.DMA,
        ],
    )
    def k(z_hbm, e0_hbm, e1_hbm, out_hbm, idx0_v, idx1_v, rows0a_v, rows1a_v,
          rows0b_v, rows1b_v, out_v, isem, gsem_a, gsem_b):
        wid = lax.axis_index("s") * nc + lax.axis_index("c")
        base = wid * b_per_w

        lane = lax.iota(jnp.int32, 16)
        perms = {k_: lane ^ k_ for k_ in (1, 2, 4, 8)}
        masks = {k_: (lane & k_) != 0 for k_ in (1, 2, 4, 8)}

        ci0 = pltpu.async_copy(e0_hbm.at[pl.ds(base, b_per_w)], idx0_v, isem)
        ci1 = pltpu.async_copy(e1_hbm.at[pl.ds(base, b_per_w)], idx1_v, isem)
        ci0.wait()
        ci1.wait()

        def launch(c, r0, r1, sem):
            pltpu.async_copy(
                z_hbm.at[idx0_v.at[pl.ds(c * _CHUNK, _CHUNK)]], r0, sem)
            pltpu.async_copy(
                z_hbm.at[idx1_v.at[pl.ds(c * _CHUNK, _CHUNK)]], r1, sem)

        def drain(c, r0, r1, sem):
            pltpu.make_async_copy(
                z_hbm.at[idx0_v.at[pl.ds(c * _CHUNK, _CHUNK)]], r0, sem).wait()
            pltpu.make_async_copy(
                z_hbm.at[idx1_v.at[pl.ds(c * _CHUNK, _CHUNK)]], r1, sem).wait()

        def dot16(r0, r1, e):
            acc = None
            for kk in range(_DW // 16):
                ws = r0[e, pl.ds(kk * 16, 16)]
                wd = r1[e, pl.ds(kk * 16, 16)]
                hs = lax.bitcast_convert_type(ws, jnp.float32)
                hd = lax.bitcast_convert_type(wd, jnp.float32)
                ls = lax.bitcast_convert_type(ws << 16, jnp.float32)
                ld = lax.bitcast_convert_type(wd << 16, jnp.float32)
                s = hs * hd + ls * ld
                acc = s if acc is None else acc + s
            return acc

        def compute(c, r0, r1):
            def group_body(g, carry):
                vecs = [dot16(r0, r1, g * _GROUP + j) for j in range(_GROUP)]
                for k_ in (1, 2, 4, 8):
                    nxt = []
                    for p in range(0, len(vecs), 2):
                        a, b = vecs[p], vecs[p + 1]
                        ra = a + _lane_gather(a, perms[k_])
                        rb = b + _lane_gather(b, perms[k_])
                        nxt.append(jnp.where(masks[k_], rb, ra))
                    vecs = nxt
                x = vecs[0]
                y = 1.0 / (1.0 + jnp.exp(-x))
                out_v[pl.ds(c * _CHUNK + g * _GROUP, _GROUP)] = y
                return carry

            lax.fori_loop(0, _CHUNK // _GROUP, group_body, 0)

        launch(0, rows0a_v, rows1a_v, gsem_a)

        def pair_body(it, carry):
            c = it * 2
            launch(c + 1, rows0b_v, rows1b_v, gsem_b)
            drain(c, rows0a_v, rows1a_v, gsem_a)
            compute(c, rows0a_v, rows1a_v)
            launch(c + 2, rows0a_v, rows1a_v, gsem_a)
            drain(c + 1, rows0b_v, rows1b_v, gsem_b)
            compute(c + 1, rows0b_v, rows1b_v)
            return carry

        lax.fori_loop(0, (n_chunks - 1) // 2, pair_body, 0)

        drain(n_chunks - 1, rows0a_v, rows1a_v, gsem_a)
        compute(n_chunks - 1, rows0a_v, rows1a_v)

        pltpu.sync_copy(out_v, out_hbm.at[pl.ds(base, b_per_w)])

    return k


def kernel(z, edge):
    z16 = z.astype(jnp.bfloat16)
    zw = lax.bitcast_convert_type(z16.reshape(z.shape[0], _DW, 2), jnp.int32)
    e0 = edge[0].astype(jnp.int32)
    e1 = edge[1].astype(jnp.int32)
    k = _build(e0.shape[0], z.shape[0])
    return k(zw, e0, e1)


# speedup vs baseline: 10.0064x; 1.0214x over previous
import functools

import jax
import jax.numpy as jnp
from jax import lax
from jax.experimental import pallas as pl
from jax.experimental.pallas import tpu as pltpu
from jax.experimental.pallas import tpu_sc as plsc

_CHUNK = 80
_GROUP = 16
_D = 128
_DW = _D // 2


def _lane_gather(a, perm):
    dnums = lax.GatherDimensionNumbers(
        offset_dims=(), collapsed_slice_dims=(0,), start_index_map=(0,))
    return lax.gather(a, perm[:, None], dnums, (1,),
                      mode=lax.GatherScatterMode.PROMISE_IN_BOUNDS)


@functools.cache
def _build(n_edges, n_rows):
    info = plsc.get_sparse_core_info()
    nc, ns = info.num_cores, info.num_subcores
    nw = nc * ns
    b_per_w = n_edges // nw
    n_chunks = b_per_w // _CHUNK
    assert n_edges % nw == 0 and b_per_w % _CHUNK == 0 and n_chunks % 2 == 1

    mesh = plsc.VectorSubcoreMesh(core_axis_name="c", subcore_axis_name="s")

    @functools.partial(
        pl.kernel,
        mesh=mesh,
        compiler_params=pltpu.CompilerParams(use_tc_tiling_on_sc=False),
        out_type=jax.ShapeDtypeStruct((n_edges,), jnp.float32),
        scratch_types=[
            pltpu.VMEM((b_per_w,), jnp.int32),
            pltpu.VMEM((b_per_w,), jnp.int32),
            pltpu.VMEM((_CHUNK, _DW), jnp.int32),
            pltpu.VMEM((_CHUNK, _DW), jnp.int32),
            pltpu.VMEM((_CHUNK, _DW), jnp.int32),
            pltpu.VMEM((_CHUNK, _DW), jnp.int32),
            pltpu.VMEM((b_per_w,), jnp.float32),
            pltpu.SemaphoreType.DMA,
            pltpu.SemaphoreType.DMA,
            pltpu.SemaphoreType.DMA,
        ],
    )
    def k(z_hbm, e0_hbm, e1_hbm, out_hbm, idx0_v, idx1_v, rows0a_v, rows1a_v,
          rows0b_v, rows1b_v, out_v, isem, gsem_a, gsem_b):
        wid = lax.axis_index("s") * nc + lax.axis_index("c")
        base = wid * b_per_w

        lane = lax.iota(jnp.int32, 16)
        perms = {k_: lane ^ k_ for k_ in (1, 2, 4, 8)}
        masks = {k_: (lane & k_) != 0 for k_ in (1, 2, 4, 8)}

        ci0 = pltpu.async_copy(e0_hbm.at[pl.ds(base, b_per_w)], idx0_v, isem)
        ci1 = pltpu.async_copy(e1_hbm.at[pl.ds(base, b_per_w)], idx1_v, isem)
        ci0.wait()
        ci1.wait()

        def launch(c, r0, r1, sem):
            pltpu.async_copy(
                z_hbm.at[idx0_v.at[pl.ds(c * _CHUNK, _CHUNK)]], r0, sem)
            pltpu.async_copy(
                z_hbm.at[idx1_v.at[pl.ds(c * _CHUNK, _CHUNK)]], r1, sem)

        def drain(c, r0, r1, sem):
            pltpu.make_async_copy(
                z_hbm.at[idx0_v.at[pl.ds(c * _CHUNK, _CHUNK)]], r0, sem).wait()
            pltpu.make_async_copy(
                z_hbm.at[idx1_v.at[pl.ds(c * _CHUNK, _CHUNK)]], r1, sem).wait()

        def dot16(r0, r1, e):
            acc = None
            for kk in range(_DW // 16):
                ws = r0[e, pl.ds(kk * 16, 16)]
                wd = r1[e, pl.ds(kk * 16, 16)]
                hs = lax.bitcast_convert_type(ws, jnp.float32)
                hd = lax.bitcast_convert_type(wd, jnp.float32)
                ls = lax.bitcast_convert_type(ws << 16, jnp.float32)
                ld = lax.bitcast_convert_type(wd << 16, jnp.float32)
                s = hs * hd + ls * ld
                acc = s if acc is None else acc + s
            return acc

        def compute(c, r0, r1):
            @plsc.parallel_loop(0, _CHUNK // _GROUP, unroll=2)
            def group_body(g):
                vecs = [dot16(r0, r1, g * _GROUP + j) for j in range(_GROUP)]
                for k_ in (1, 2, 4, 8):
                    nxt = []
                    for p in range(0, len(vecs), 2):
                        a, b = vecs[p], vecs[p + 1]
                        ra = a + _lane_gather(a, perms[k_])
                        rb = b + _lane_gather(b, perms[k_])
                        nxt.append(jnp.where(masks[k_], rb, ra))
                    vecs = nxt
                x = vecs[0]
                y = 1.0 / (1.0 + jnp.exp(-x))
                out_v[pl.ds(c * _CHUNK + g * _GROUP, _GROUP)] = y

        launch(0, rows0a_v, rows1a_v, gsem_a)

        def pair_body(it, carry):
            c = it * 2
            launch(c + 1, rows0b_v, rows1b_v, gsem_b)
            drain(c, rows0a_v, rows1a_v, gsem_a)
            compute(c, rows0a_v, rows1a_v)
            launch(c + 2, rows0a_v, rows1a_v, gsem_a)
            drain(c + 1, rows0b_v, rows1b_v, gsem_b)
            compute(c + 1, rows0b_v, rows1b_v)
            return carry

        lax.fori_loop(0, (n_chunks - 1) // 2, pair_body, 0)

        drain(n_chunks - 1, rows0a_v, rows1a_v, gsem_a)
        compute(n_chunks - 1, rows0a_v, rows1a_v)

        pltpu.sync_copy(out_v, out_hbm.at[pl.ds(base, b_per_w)])

    return k


def kernel(z, edge):
    z16 = z.astype(jnp.bfloat16)
    zw = lax.bitcast_convert_type(z16.reshape(z.shape[0], _DW, 2), jnp.int32)
    e0 = edge[0].astype(jnp.int32)
    e1 = edge[1].astype(jnp.int32)
    k = _build(e0.shape[0], z.shape[0])
    return k(zw, e0, e1)


# speedup vs baseline: 11.0278x; 1.1021x over previous
import functools

import jax
import jax.numpy as jnp
from jax import lax
from jax.experimental import pallas as pl
from jax.experimental.pallas import tpu as pltpu
from jax.experimental.pallas import tpu_sc as plsc

_CHUNK = 128
_GROUP = 16
_D = 128
_DW = _D // 2


def _lane_gather(a, perm):
    dnums = lax.GatherDimensionNumbers(
        offset_dims=(), collapsed_slice_dims=(0,), start_index_map=(0,))
    return lax.gather(a, perm[:, None], dnums, (1,),
                      mode=lax.GatherScatterMode.PROMISE_IN_BOUNDS)


@functools.cache
def _build(n_edges, n_rows):
    info = plsc.get_sparse_core_info()
    nc, ns = info.num_cores, info.num_subcores
    nw = nc * ns
    b_per_w = n_edges // nw
    n_full = b_per_w // _CHUNK
    tail = b_per_w - n_full * _CHUNK
    assert n_edges % nw == 0 and n_full % 2 == 0 and tail % _GROUP == 0
    assert tail % 8 == 0 and tail > 0

    mesh = plsc.VectorSubcoreMesh(core_axis_name="c", subcore_axis_name="s")

    @functools.partial(
        pl.kernel,
        mesh=mesh,
        compiler_params=pltpu.CompilerParams(use_tc_tiling_on_sc=False),
        out_type=jax.ShapeDtypeStruct((n_edges,), jnp.float32),
        scratch_types=[
            pltpu.VMEM((b_per_w,), jnp.int32),
            pltpu.VMEM((b_per_w,), jnp.int32),
            pltpu.VMEM((_CHUNK, _DW), jnp.int32),
            pltpu.VMEM((_CHUNK, _DW), jnp.int32),
            pltpu.VMEM((_CHUNK, _DW), jnp.int32),
            pltpu.VMEM((_CHUNK, _DW), jnp.int32),
            pltpu.VMEM((b_per_w,), jnp.float32),
            pltpu.SemaphoreType.DMA,
            pltpu.SemaphoreType.DMA,
            pltpu.SemaphoreType.DMA,
        ],
    )
    def k(z_hbm, e0_hbm, e1_hbm, out_hbm, idx0_v, idx1_v, rows0a_v, rows1a_v,
          rows0b_v, rows1b_v, out_v, isem, gsem_a, gsem_b):
        wid = lax.axis_index("s") * nc + lax.axis_index("c")
        base = wid * b_per_w

        lane = lax.iota(jnp.int32, 16)
        perms = {k_: lane ^ k_ for k_ in (1, 2, 4, 8)}
        masks = {k_: (lane & k_) != 0 for k_ in (1, 2, 4, 8)}

        ci0 = pltpu.async_copy(e0_hbm.at[pl.ds(base, b_per_w)], idx0_v, isem)
        ci1 = pltpu.async_copy(e1_hbm.at[pl.ds(base, b_per_w)], idx1_v, isem)
        ci0.wait()
        ci1.wait()

        def launch(c, r0, r1, sem, n=_CHUNK):
            pltpu.async_copy(
                z_hbm.at[idx0_v.at[pl.ds(c * _CHUNK, n)]],
                r0.at[pl.ds(0, n)], sem)
            pltpu.async_copy(
                z_hbm.at[idx1_v.at[pl.ds(c * _CHUNK, n)]],
                r1.at[pl.ds(0, n)], sem)

        def drain(c, r0, r1, sem, n=_CHUNK):
            pltpu.make_async_copy(
                z_hbm.at[idx0_v.at[pl.ds(c * _CHUNK, n)]],
                r0.at[pl.ds(0, n)], sem).wait()
            pltpu.make_async_copy(
                z_hbm.at[idx1_v.at[pl.ds(c * _CHUNK, n)]],
                r1.at[pl.ds(0, n)], sem).wait()

        def dot16(r0, r1, e):
            acc = None
            for kk in range(_DW // 16):
                ws = r0[e, pl.ds(kk * 16, 16)]
                wd = r1[e, pl.ds(kk * 16, 16)]
                hs = lax.bitcast_convert_type(ws, jnp.float32)
                hd = lax.bitcast_convert_type(wd, jnp.float32)
                ls = lax.bitcast_convert_type(ws << 16, jnp.float32)
                ld = lax.bitcast_convert_type(wd << 16, jnp.float32)
                s = hs * hd + ls * ld
                acc = s if acc is None else acc + s
            return acc

        def compute(c, r0, r1, ngroups=_CHUNK // _GROUP):
            @plsc.parallel_loop(0, ngroups, unroll=2)
            def group_body(g):
                vecs = [dot16(r0, r1, g * _GROUP + j) for j in range(_GROUP)]
                for k_ in (1, 2, 4, 8):
                    nxt = []
                    for p in range(0, len(vecs), 2):
                        a, b = vecs[p], vecs[p + 1]
                        ra = a + _lane_gather(a, perms[k_])
                        rb = b + _lane_gather(b, perms[k_])
                        nxt.append(jnp.where(masks[k_], rb, ra))
                    vecs = nxt
                x = vecs[0]
                y = 1.0 / (1.0 + jnp.exp(-x))
                out_v[pl.ds(c * _CHUNK + g * _GROUP, _GROUP)] = y

        launch(0, rows0a_v, rows1a_v, gsem_a)

        def pair_body(it, carry):
            c = it * 2
            launch(c + 1, rows0b_v, rows1b_v, gsem_b)
            drain(c, rows0a_v, rows1a_v, gsem_a)
            compute(c, rows0a_v, rows1a_v)

            @pl.when(c + 2 < n_full)
            def _():
                launch(c + 2, rows0a_v, rows1a_v, gsem_a)

            @pl.when(c + 2 == n_full)
            def _():
                launch(n_full, rows0a_v, rows1a_v, gsem_a, n=tail)

            drain(c + 1, rows0b_v, rows1b_v, gsem_b)
            compute(c + 1, rows0b_v, rows1b_v)
            return carry

        lax.fori_loop(0, n_full // 2, pair_body, 0)

        drain(n_full, rows0a_v, rows1a_v, gsem_a, n=tail)
        compute(n_full, rows0a_v, rows1a_v, ngroups=tail // _GROUP)

        pltpu.sync_copy(out_v, out_hbm.at[pl.ds(base, b_per_w)])

    return k


def kernel(z, edge):
    z16 = z.astype(jnp.bfloat16)
    zw = lax.bitcast_convert_type(z16.reshape(z.shape[0], _DW, 2), jnp.int32)
    e0 = edge[0].astype(jnp.int32)
    e1 = edge[1].astype(jnp.int32)
    k = _build(e0.shape[0], z.shape[0])
    return k(zw, e0, e1)


# speedup vs baseline: 11.1244x; 1.0088x over previous
import functools

import jax
import jax.numpy as jnp
from jax import lax
from jax.experimental import pallas as pl
from jax.experimental.pallas import tpu as pltpu
from jax.experimental.pallas import tpu_sc as plsc

_CHUNK = 128
_GROUP = 16
_D = 128
_DW = _D // 2


def _lane_gather(a, perm):
    dnums = lax.GatherDimensionNumbers(
        offset_dims=(), collapsed_slice_dims=(0,), start_index_map=(0,))
    return lax.gather(a, perm[:, None], dnums, (1,),
                      mode=lax.GatherScatterMode.PROMISE_IN_BOUNDS)


@functools.cache
def _build(n_edges, n_rows):
    info = plsc.get_sparse_core_info()
    nc, ns = info.num_cores, info.num_subcores
    nw = nc * ns
    b_per_w = n_edges // nw
    n_full = b_per_w // _CHUNK
    tail = b_per_w - n_full * _CHUNK
    assert n_edges % nw == 0 and n_full % 2 == 0 and tail % _GROUP == 0
    assert tail % 8 == 0 and tail > 0

    mesh = plsc.VectorSubcoreMesh(core_axis_name="c", subcore_axis_name="s")

    @functools.partial(
        pl.kernel,
        mesh=mesh,
        compiler_params=pltpu.CompilerParams(use_tc_tiling_on_sc=False),
        out_type=jax.ShapeDtypeStruct((n_edges,), jnp.float32),
        scratch_types=[
            pltpu.VMEM((b_per_w,), jnp.int32),
            pltpu.VMEM((b_per_w,), jnp.int32),
            pltpu.VMEM((_CHUNK, _DW), jnp.int32),
            pltpu.VMEM((_CHUNK, _DW), jnp.int32),
            pltpu.VMEM((_CHUNK, _DW), jnp.int32),
            pltpu.VMEM((_CHUNK, _DW), jnp.int32),
            pltpu.VMEM((b_per_w,), jnp.float32),
            pltpu.SemaphoreType.DMA,
            pltpu.SemaphoreType.DMA,
            pltpu.SemaphoreType.DMA,
        ],
    )
    def k(z_hbm, e0_hbm, e1_hbm, out_hbm, idx0_v, idx1_v, rows0a_v, rows1a_v,
          rows0b_v, rows1b_v, out_v, isem, gsem_a, gsem_b):
        wid = lax.axis_index("s") * nc + lax.axis_index("c")
        base = wid * b_per_w

        lane = lax.iota(jnp.int32, 16)
        perms = {k_: lane ^ k_ for k_ in (1, 2, 4, 8)}
        masks = {k_: (lane & k_) != 0 for k_ in (1, 2, 4, 8)}

        ci0 = pltpu.async_copy(e0_hbm.at[pl.ds(base, b_per_w)], idx0_v, isem)
        ci1 = pltpu.async_copy(e1_hbm.at[pl.ds(base, b_per_w)], idx1_v, isem)
        ci0.wait()
        ci1.wait()

        def launch(c, r0, r1, sem, n=_CHUNK):
            pltpu.async_copy(
                z_hbm.at[idx0_v.at[pl.ds(c * _CHUNK, n)]],
                r0.at[pl.ds(0, n)], sem)
            pltpu.async_copy(
                z_hbm.at[idx1_v.at[pl.ds(c * _CHUNK, n)]],
                r1.at[pl.ds(0, n)], sem)

        def drain(c, r0, r1, sem, n=_CHUNK):
            pltpu.make_async_copy(
                z_hbm.at[idx0_v.at[pl.ds(c * _CHUNK, n)]],
                r0.at[pl.ds(0, n)], sem).wait()
            pltpu.make_async_copy(
                z_hbm.at[idx1_v.at[pl.ds(c * _CHUNK, n)]],
                r1.at[pl.ds(0, n)], sem).wait()

        def dot16(r0, r1, e):
            acc = None
            for kk in range(_DW // 16):
                ws = r0[e, pl.ds(kk * 16, 16)]
                wd = r1[e, pl.ds(kk * 16, 16)]
                hs = lax.bitcast_convert_type(ws, jnp.float32)
                hd = lax.bitcast_convert_type(wd, jnp.float32)
                ls = lax.bitcast_convert_type(ws << 16, jnp.float32)
                ld = lax.bitcast_convert_type(wd << 16, jnp.float32)
                s = hs * hd + ls * ld
                acc = s if acc is None else acc + s
            return acc

        def compute(c, r0, r1, ngroups=_CHUNK // _GROUP):
            @plsc.parallel_loop(0, ngroups, unroll=4)
            def group_body(g):
                vecs = [dot16(r0, r1, g * _GROUP + j) for j in range(_GROUP)]
                for k_ in (1, 2, 4, 8):
                    nxt = []
                    for p in range(0, len(vecs), 2):
                        a, b = vecs[p], vecs[p + 1]
                        ra = a + _lane_gather(a, perms[k_])
                        rb = b + _lane_gather(b, perms[k_])
                        nxt.append(jnp.where(masks[k_], rb, ra))
                    vecs = nxt
                x = vecs[0]
                y = 1.0 / (1.0 + jnp.exp(-x))
                out_v[pl.ds(c * _CHUNK + g * _GROUP, _GROUP)] = y

        launch(0, rows0a_v, rows1a_v, gsem_a)

        def pair_body(it, carry):
            c = it * 2
            launch(c + 1, rows0b_v, rows1b_v, gsem_b)
            drain(c, rows0a_v, rows1a_v, gsem_a)
            compute(c, rows0a_v, rows1a_v)

            @pl.when(c + 2 < n_full)
            def _():
                launch(c + 2, rows0a_v, rows1a_v, gsem_a)

            @pl.when(c + 2 == n_full)
            def _():
                launch(n_full, rows0a_v, rows1a_v, gsem_a, n=tail)

            drain(c + 1, rows0b_v, rows1b_v, gsem_b)
            compute(c + 1, rows0b_v, rows1b_v)
            return carry

        lax.fori_loop(0, n_full // 2, pair_body, 0)

        drain(n_full, rows0a_v, rows1a_v, gsem_a, n=tail)
        compute(n_full, rows0a_v, rows1a_v, ngroups=tail // _GROUP)

        pltpu.sync_copy(out_v, out_hbm.at[pl.ds(base, b_per_w)])

    return k


def kernel(z, edge):
    z16 = z.astype(jnp.bfloat16)
    zw = lax.bitcast_convert_type(z16.reshape(z.shape[0], _DW, 2), jnp.int32)
    e0 = edge[0].astype(jnp.int32)
    e1 = edge[1].astype(jnp.int32)
    k = _build(e0.shape[0], z.shape[0])
    return k(zw, e0, e1)


# speedup vs baseline: 12.4476x; 1.1189x over previous
import functools

import jax
import jax.numpy as jnp
from jax import lax
from jax.experimental import pallas as pl
from jax.experimental.pallas import tpu as pltpu
from jax.experimental.pallas import tpu_sc as plsc

_CHUNK = 128
_GROUP = 16
_D = 128
_DW = _D // 2


def _lane_gather(a, perm):
    dnums = lax.GatherDimensionNumbers(
        offset_dims=(), collapsed_slice_dims=(0,), start_index_map=(0,))
    return lax.gather(a, perm[:, None], dnums, (1,),
                      mode=lax.GatherScatterMode.PROMISE_IN_BOUNDS)


@functools.cache
def _build(n_edges, n_rows):
    info = plsc.get_sparse_core_info()
    nc, ns = info.num_cores, info.num_subcores
    nw = nc * ns
    b_per_w = n_edges // nw
    n_full = b_per_w // _CHUNK
    tail = b_per_w - n_full * _CHUNK
    assert n_edges % nw == 0 and n_full % 2 == 0 and tail % _GROUP == 0
    assert tail % 8 == 0 and tail > 0

    mesh = plsc.VectorSubcoreMesh(core_axis_name="c", subcore_axis_name="s")

    @functools.partial(
        pl.kernel,
        mesh=mesh,
        compiler_params=pltpu.CompilerParams(use_tc_tiling_on_sc=False),
        out_type=jax.ShapeDtypeStruct((n_edges,), jnp.float32),
        scratch_types=[
            pltpu.VMEM((b_per_w,), jnp.int32),
            pltpu.VMEM((b_per_w,), jnp.int32),
            pltpu.VMEM((_CHUNK, _DW), jnp.int32),
            pltpu.VMEM((_CHUNK, _DW), jnp.int32),
            pltpu.VMEM((_CHUNK, _DW), jnp.int32),
            pltpu.VMEM((_CHUNK, _DW), jnp.int32),
            pltpu.VMEM((_CHUNK, _DW), jnp.int32),
            pltpu.VMEM((_CHUNK, _DW), jnp.int32),
            pltpu.VMEM((b_per_w,), jnp.float32),
            pltpu.SemaphoreType.DMA,
            pltpu.SemaphoreType.DMA,
            pltpu.SemaphoreType.DMA,
            pltpu.SemaphoreType.DMA,
        ],
    )
    def k(z_hbm, e0_hbm, e1_hbm, out_hbm, idx0_v, idx1_v, rows0a_v, rows1a_v,
          rows0b_v, rows1b_v, rows0c_v, rows1c_v, out_v, isem, gsem_a, gsem_b,
          gsem_c):
        wid = lax.axis_index("s") * nc + lax.axis_index("c")
        base = wid * b_per_w

        lane = lax.iota(jnp.int32, 16)
        perms = {k_: lane ^ k_ for k_ in (1, 2, 4, 8)}
        masks = {k_: (lane & k_) != 0 for k_ in (1, 2, 4, 8)}

        ci0 = pltpu.async_copy(e0_hbm.at[pl.ds(base, b_per_w)], idx0_v, isem)
        ci1 = pltpu.async_copy(e1_hbm.at[pl.ds(base, b_per_w)], idx1_v, isem)
        ci0.wait()
        ci1.wait()

        def launch(c, r0, r1, sem, n=_CHUNK):
            pltpu.async_copy(
                z_hbm.at[idx0_v.at[pl.ds(c * _CHUNK, n)]],
                r0.at[pl.ds(0, n)], sem)
            pltpu.async_copy(
                z_hbm.at[idx1_v.at[pl.ds(c * _CHUNK, n)]],
                r1.at[pl.ds(0, n)], sem)

        def drain(c, r0, r1, sem, n=_CHUNK):
            pltpu.make_async_copy(
                z_hbm.at[idx0_v.at[pl.ds(c * _CHUNK, n)]],
                r0.at[pl.ds(0, n)], sem).wait()
            pltpu.make_async_copy(
                z_hbm.at[idx1_v.at[pl.ds(c * _CHUNK, n)]],
                r1.at[pl.ds(0, n)], sem).wait()

        def dot16(r0, r1, e):
            acc = None
            for kk in range(_DW // 16):
                ws = r0[e, pl.ds(kk * 16, 16)]
                wd = r1[e, pl.ds(kk * 16, 16)]
                hs = lax.bitcast_convert_type(ws, jnp.float32)
                hd = lax.bitcast_convert_type(wd, jnp.float32)
                ls = lax.bitcast_convert_type(ws << 16, jnp.float32)
                ld = lax.bitcast_convert_type(wd << 16, jnp.float32)
                s = hs * hd + ls * ld
                acc = s if acc is None else acc + s
            return acc

        def compute(c, r0, r1, ngroups=_CHUNK // _GROUP):
            @plsc.parallel_loop(0, ngroups, unroll=2)
            def group_body(g):
                vecs = [dot16(r0, r1, g * _GROUP + j) for j in range(_GROUP)]
                for k_ in (1, 2, 4, 8):
                    nxt = []
                    for p in range(0, len(vecs), 2):
                        a, b = vecs[p], vecs[p + 1]
                        ra = a + _lane_gather(a, perms[k_])
                        rb = b + _lane_gather(b, perms[k_])
                        nxt.append(jnp.where(masks[k_], rb, ra))
                    vecs = nxt
                x = vecs[0]
                y = 1.0 / (1.0 + jnp.exp(-x))
                out_v[pl.ds(c * _CHUNK + g * _GROUP, _GROUP)] = y

        bufs = ((rows0a_v, rows1a_v, gsem_a),
                (rows0b_v, rows1b_v, gsem_b),
                (rows0c_v, rows1c_v, gsem_c))
        assert n_full % 3 == 0

        launch(0, *bufs[0])
        launch(1, *bufs[1])

        def trio_body(it, carry):
            for b in range(3):
                c = it * 3 + b
                r0, r1, sem = bufs[b]
                nb = bufs[(b + 2) % 3]
                drain(c, r0, r1, sem)

                @pl.when(c + 2 < n_full)
                def _():
                    launch(c + 2, *nb)

                @pl.when(c + 2 == n_full)
                def _():
                    launch(n_full, nb[0], nb[1], nb[2], n=tail)

                compute(c, r0, r1)
            return carry

        lax.fori_loop(0, n_full // 3, trio_body, 0)

        tb = bufs[n_full % 3]
        drain(n_full, tb[0], tb[1], tb[2], n=tail)
        compute(n_full, tb[0], tb[1], ngroups=tail // _GROUP)

        pltpu.sync_copy(out_v, out_hbm.at[pl.ds(base, b_per_w)])

    return k


def kernel(z, edge):
    z16 = z.astype(jnp.bfloat16)
    zw = lax.bitcast_convert_type(z16.reshape(z.shape[0], _DW, 2), jnp.int32)
    e0 = edge[0].astype(jnp.int32)
    e1 = edge[1].astype(jnp.int32)
    k = _build(e0.shape[0], z.shape[0])
    return k(zw, e0, e1)
